# Initial kernel scaffold; baseline (speedup 1.0000x reference)
import jax
import jax.numpy as jnp
from jax.experimental import pallas as pl


def kernel(q, k, idx):
    raise NotImplementedError("write your pallas kernel here")


# speedup vs baseline: 91.7377x; 91.7377x over previous
import functools

import jax
import jax.numpy as jnp
from jax import lax
from jax.experimental import pallas as pl
from jax.experimental.pallas import tpu as pltpu
from jax.experimental.pallas import tpu_sc as plsc

_B, _H, _S, _D, _K = 1, 16, 2048, 128, 64
_R = _H * _S
_BQ = 512

_NC, _NS = 2, 16
_NW = _NC * _NS
_RPW = _R // _NW
_CH = 16
_NCHUNK = _RPW // _CH


def _tc_scores_body(q_ref, k_ref, out_ref):
    q = q_ref[0]
    kk = k_ref[0]
    out_ref[...] = lax.dot_general(
        q, kk, (((1,), (1,)), ((), ())), preferred_element_type=jnp.float32
    )


def _tc_scores(q3, k3):
    return pl.pallas_call(
        _tc_scores_body,
        grid=(_H, _S // _BQ),
        in_specs=[
            pl.BlockSpec((1, _BQ, _D), lambda h, qb: (h, qb, 0)),
            pl.BlockSpec((1, _S, _D), lambda h, qb: (h, 0, 0)),
        ],
        out_specs=pl.BlockSpec((_BQ, _S), lambda h, qb: (h * (_S // _BQ) + qb, 0)),
        out_shape=jax.ShapeDtypeStruct((_R, _S), jnp.float32),
    )(q3, k3)


def _bf16_bits_rne(v32):
    b = plsc.bitcast(v32, jnp.int32)
    b = b + (lax.shift_right_logical(b, 16) & 1) + 0x7FFF
    return lax.shift_right_logical(b, 16)


def _sc_gather_body(scores_hbm, idx_hbm, out_hbm, sc_v, idx_v, out_v):
    wid = lax.axis_index("s") * _NC + lax.axis_index("c")
    row0 = wid * _RPW
    iota = lax.iota(jnp.int32, 16)

    def chunk(g, carry):
        base = row0 + g * _CH
        pltpu.sync_copy(scores_hbm.at[pl.ds(base, _CH)], sc_v)
        pltpu.sync_copy(idx_hbm.at[pl.ds(base, _CH)], idx_v)
        for r in range(_CH):
            rr = jnp.full((16,), r, jnp.int32)
            for half in range(2):
                ce = jnp.full((16,), 32 * half, jnp.int32) + 2 * iota
                ie = plsc.load_gather(idx_v, [rr, ce])
                io = plsc.load_gather(idx_v, [rr, ce + 1])
                ve = plsc.load_gather(sc_v, [rr, ie])
                vo = plsc.load_gather(sc_v, [rr, io])
                we = _bf16_bits_rne(ve)
                wo = _bf16_bits_rne(vo)
                out_v[r, pl.ds(16 * half, 16)] = we | lax.shift_left(wo, 16)
        pltpu.sync_copy(out_v, out_hbm.at[pl.ds(base, _CH)])
        return carry

    lax.fori_loop(0, _NCHUNK, chunk, 0)


def _sc_gather(scores, idx2):
    mesh = plsc.VectorSubcoreMesh(core_axis_name="c", subcore_axis_name="s")
    fn = functools.partial(
        pl.kernel,
        mesh=mesh,
        compiler_params=pltpu.CompilerParams(needs_layout_passes=False),
        out_type=jax.ShapeDtypeStruct((_R, _K // 2), jnp.int32),
        scratch_types=[
            pltpu.VMEM((_CH, _S), jnp.float32),
            pltpu.VMEM((_CH, _K), jnp.int32),
            pltpu.VMEM((_CH, _K // 2), jnp.int32),
        ],
    )(_sc_gather_body)
    return fn(scores, idx2)


@jax.jit
def kernel(q, k, idx):
    q3 = q.reshape(_H, _S, _D)
    k3 = k.reshape(_H, _S, _D)
    idx2 = idx.reshape(_R, _K)
    scores = _tc_scores(q3, k3)
    out32 = _sc_gather(scores, idx2)
    out = lax.bitcast_convert_type(out32, jnp.bfloat16)
    return out.reshape(_B, _H, _S, _K)


# speedup vs baseline: 95.9672x; 1.0461x over previous
import functools

import jax
import jax.numpy as jnp
from jax import lax
from jax.experimental import pallas as pl
from jax.experimental.pallas import tpu as pltpu
from jax.experimental.pallas import tpu_sc as plsc

_B, _H, _S, _D, _K = 1, 16, 2048, 128, 64
_R = _H * _S
_SW = _S // 2
_BQ = 512

_NC, _NS = 2, 16
_NW = _NC * _NS
_RPW = _R // _NW
_CH = 32
_NCHUNK = _RPW // _CH


def _rne16(b):
    b = b + (lax.shift_right_logical(b, 16) & 1) + 0x7FFF
    return lax.shift_right_logical(b, 16)


def _tc_scores_body(q_ref, ke_ref, ko_ref, out_ref):
    q = q_ref[0]
    dn = (((1,), (1,)), ((), ()))
    se = lax.dot_general(q, ke_ref[0], dn, preferred_element_type=jnp.float32)
    so = lax.dot_general(q, ko_ref[0], dn, preferred_element_type=jnp.float32)
    we = _rne16(lax.bitcast_convert_type(se, jnp.int32))
    wo = _rne16(lax.bitcast_convert_type(so, jnp.int32))
    out_ref[...] = we | lax.shift_left(wo, 16)


def _tc_scores(q3, ke3, ko3):
    return pl.pallas_call(
        _tc_scores_body,
        grid=(_H, _S // _BQ),
        in_specs=[
            pl.BlockSpec((1, _BQ, _D), lambda h, qb: (h, qb, 0)),
            pl.BlockSpec((1, _SW, _D), lambda h, qb: (h, 0, 0)),
            pl.BlockSpec((1, _SW, _D), lambda h, qb: (h, 0, 0)),
        ],
        out_specs=pl.BlockSpec((_BQ, _SW), lambda h, qb: (h * (_S // _BQ) + qb, 0)),
        out_shape=jax.ShapeDtypeStruct((_R, _SW), jnp.int32),
    )(q3, ke3, ko3)


def _sc_gather_body(scores_hbm, idx_hbm, out_hbm, sc_v, idx_v, out_v):
    wid = lax.axis_index("s") * _NC + lax.axis_index("c")
    row0 = wid * _RPW
    iota = lax.iota(jnp.int32, 16)

    def chunk(g, carry):
        base = row0 + g * _CH
        pltpu.sync_copy(scores_hbm.at[pl.ds(base, _CH)], sc_v)
        pltpu.sync_copy(idx_hbm.at[pl.ds(base, _CH)], idx_v)
        for r in range(_CH):
            rr = jnp.full((16,), r, jnp.int32)
            for half in range(2):
                ce = jnp.full((16,), 32 * half, jnp.int32) + 2 * iota
                ie = plsc.load_gather(idx_v, [rr, ce])
                io = plsc.load_gather(idx_v, [rr, ce + 1])
                ge = plsc.load_gather(sc_v, [rr, lax.shift_right_logical(ie, 1)])
                go = plsc.load_gather(sc_v, [rr, lax.shift_right_logical(io, 1)])
                ve = lax.shift_right_logical(ge, lax.shift_left(ie & 1, 4)) & 0xFFFF
                vo = lax.shift_right_logical(go, lax.shift_left(io & 1, 4)) & 0xFFFF
                out_v[r, pl.ds(16 * half, 16)] = ve | lax.shift_left(vo, 16)
        pltpu.sync_copy(out_v, out_hbm.at[pl.ds(base, _CH)])
        return carry

    lax.fori_loop(0, _NCHUNK, chunk, 0)


def _sc_gather(scores, idx2):
    mesh = plsc.VectorSubcoreMesh(core_axis_name="c", subcore_axis_name="s")
    fn = functools.partial(
        pl.kernel,
        mesh=mesh,
        compiler_params=pltpu.CompilerParams(needs_layout_passes=False),
        out_type=jax.ShapeDtypeStruct((_R, _K // 2), jnp.int32),
        scratch_types=[
            pltpu.VMEM((_CH, _SW), jnp.int32),
            pltpu.VMEM((_CH, _K), jnp.int32),
            pltpu.VMEM((_CH, _K // 2), jnp.int32),
        ],
    )(_sc_gather_body)
    return fn(scores, idx2)


@jax.jit
def kernel(q, k, idx):
    q3 = q.reshape(_H, _S, _D)
    k3 = k.reshape(_H, _S, _D)
    ke3 = k3[:, 0::2, :]
    ko3 = k3[:, 1::2, :]
    idx2 = idx.reshape(_R, _K)
    scores = _tc_scores(q3, ke3, ko3)
    out32 = _sc_gather(scores, idx2)
    out = lax.bitcast_convert_type(out32, jnp.bfloat16)
    return out.reshape(_B, _H, _S, _K)


# speedup vs baseline: 163.8581x; 1.7074x over previous
import functools

import jax
import jax.numpy as jnp
from jax import lax
from jax.experimental import pallas as pl
from jax.experimental.pallas import tpu as pltpu
from jax.experimental.pallas import tpu_sc as plsc

_B, _H, _S, _D, _K = 1, 16, 2048, 128, 64
_R = _H * _S
_RP = _R // 2
_BQ = 512

_NC, _NS = 2, 16
_NW = _NC * _NS
_PPW = _RP // _NW
_CH = 16
_NCHUNK = _PPW // _CH


def _tc_scores_body(q_ref, k_ref, out_ref):
    q = q_ref[0]
    dn = (((1,), (1,)), ((), ()))
    s = lax.dot_general(q, k_ref[0], dn, preferred_element_type=jnp.float32)
    out_ref[...] = pltpu.bitcast(s.astype(jnp.bfloat16), jnp.int32)


def _tc_scores(q3, k3):
    return pl.pallas_call(
        _tc_scores_body,
        grid=(_H, _S // _BQ),
        in_specs=[
            pl.BlockSpec((1, _BQ, _D), lambda h, qb: (h, qb, 0)),
            pl.BlockSpec((1, _S, _D), lambda h, qb: (h, 0, 0)),
        ],
        out_specs=pl.BlockSpec(
            (_BQ // 2, _S), lambda h, qb: (h * (_S // _BQ) + qb, 0)
        ),
        out_shape=jax.ShapeDtypeStruct((_RP, _S), jnp.int32),
    )(q3, k3)


def _sc_gather_body(scores_hbm, idx_hbm, out_hbm,
                    sc0, sc1, ix0, ix1, ov0, ov1, si0, si1, so0, so1):
    wid = lax.axis_index("s") * _NC + lax.axis_index("c")
    p0w = wid * _PPW
    iota = lax.iota(jnp.int32, 16)
    scb, ixb, ovb = (sc0, sc1), (ix0, ix1), (ov0, ov1)
    sib, sob = (si0, si1), (so0, so1)

    def start_in(g, b):
        p0 = p0w + g * _CH
        pltpu.make_async_copy(
            scores_hbm.at[pl.ds(p0, _CH)], scb[b], sib[b]).start()
        pltpu.make_async_copy(
            idx_hbm.at[pl.ds(2 * p0, 2 * _CH)], ixb[b], sib[b]).start()

    def wait_in(b):
        pltpu.make_async_copy(
            scores_hbm.at[pl.ds(0, _CH)], scb[b], sib[b]).wait()
        pltpu.make_async_copy(
            idx_hbm.at[pl.ds(0, 2 * _CH)], ixb[b], sib[b]).wait()

    def start_out(g, b):
        p0 = p0w + g * _CH
        pltpu.make_async_copy(
            ovb[b], out_hbm.at[pl.ds(2 * p0, 2 * _CH)], sob[b]).start()

    def wait_out(b):
        pltpu.make_async_copy(
            ovb[b], out_hbm.at[pl.ds(0, 2 * _CH)], sob[b]).wait()

    for b in range(2):
        start_in(b, b)

    def outer(g2, carry):
        for b in range(2):
            g = 2 * g2 + b

            @pl.when(g2 >= 1)
            def _():
                wait_out(b)

            wait_in(b)
            sc_v, idx_v, out_v = scb[b], ixb[b], ovb[b]
            for r in range(_CH):
                rr = jnp.full((16,), r, jnp.int32)
                for par in range(2):
                    qrow = jnp.full((16,), 2 * r + par, jnp.int32)
                    for half in range(2):
                        ce = jnp.full((16,), 32 * half, jnp.int32) + 2 * iota
                        ie = plsc.load_gather(idx_v, [qrow, ce])
                        io = plsc.load_gather(idx_v, [qrow, ce + 1])
                        ge = plsc.load_gather(sc_v, [rr, ie])
                        go = plsc.load_gather(sc_v, [rr, io])
                        if par == 0:
                            w = (ge & 0xFFFF) | lax.shift_left(go, 16)
                        else:
                            w = lax.shift_right_logical(ge, 16) | (
                                go & jnp.int32(-65536))
                        out_v[2 * r + par, pl.ds(16 * half, 16)] = w

            start_out(g, b)

            @pl.when(g2 < _NCHUNK // 2 - 1)
            def _():
                start_in(g + 2, b)

        return carry

    lax.fori_loop(0, _NCHUNK // 2, outer, 0)
    for b in range(2):
        wait_out(b)


def _sc_gather(scores, idx2):
    mesh = plsc.VectorSubcoreMesh(core_axis_name="c", subcore_axis_name="s")
    fn = functools.partial(
        pl.kernel,
        mesh=mesh,
        compiler_params=pltpu.CompilerParams(needs_layout_passes=False),
        out_type=jax.ShapeDtypeStruct((_R, _K // 2), jnp.int32),
        scratch_types=[
            pltpu.VMEM((_CH, _S), jnp.int32),
            pltpu.VMEM((_CH, _S), jnp.int32),
            pltpu.VMEM((2 * _CH, _K), jnp.int32),
            pltpu.VMEM((2 * _CH, _K), jnp.int32),
            pltpu.VMEM((2 * _CH, _K // 2), jnp.int32),
            pltpu.VMEM((2 * _CH, _K // 2), jnp.int32),
            pltpu.SemaphoreType.DMA,
            pltpu.SemaphoreType.DMA,
            pltpu.SemaphoreType.DMA,
            pltpu.SemaphoreType.DMA,
        ],
    )(_sc_gather_body)
    return fn(scores, idx2)


@jax.jit
def kernel(q, k, idx):
    q3 = q.reshape(_H, _S, _D)
    k3 = k.reshape(_H, _S, _D)
    idx2 = idx.reshape(_R, _K)
    scores = _tc_scores(q3, k3)
    out32 = _sc_gather(scores, idx2)
    out = lax.bitcast_convert_type(out32, jnp.bfloat16)
    return out.reshape(_B, _H, _S, _K)


# speedup vs baseline: 164.6845x; 1.0050x over previous
import functools

import jax
import jax.numpy as jnp
from jax import lax
from jax.experimental import pallas as pl
from jax.experimental.pallas import tpu as pltpu
from jax.experimental.pallas import tpu_sc as plsc

_B, _H, _S, _D, _K = 1, 16, 2048, 128, 64
_R = _H * _S
_BQ = 512
_G = 2
_HG = _H // _G
_RG = _HG * _S
_RPG = _RG // 2

_NC, _NS = 2, 16
_NW = _NC * _NS
_PPW = _RPG // _NW
_CH = 16
_NCHUNK = _PPW // _CH


def _tc_scores_body(q_ref, k_ref, out_ref):
    q = q_ref[0]
    dn = (((1,), (1,)), ((), ()))
    s = lax.dot_general(q, k_ref[0], dn, preferred_element_type=jnp.float32)
    out_ref[...] = pltpu.bitcast(s.astype(jnp.bfloat16), jnp.int32)


def _tc_scores(q3, k3):
    return pl.pallas_call(
        _tc_scores_body,
        grid=(_HG, _S // _BQ),
        in_specs=[
            pl.BlockSpec((1, _BQ, _D), lambda h, qb: (h, qb, 0)),
            pl.BlockSpec((1, _S, _D), lambda h, qb: (h, 0, 0)),
        ],
        out_specs=pl.BlockSpec(
            (_BQ // 2, _S), lambda h, qb: (h * (_S // _BQ) + qb, 0)
        ),
        out_shape=jax.ShapeDtypeStruct((_RPG, _S), jnp.int32),
    )(q3, k3)


def _sc_gather_body(scores_hbm, idx_hbm, out_hbm,
                    sc0, sc1, ix0, ix1, ov0, ov1, si0, si1, so0, so1):
    wid = lax.axis_index("s") * _NC + lax.axis_index("c")
    p0w = wid * _PPW
    iota = lax.iota(jnp.int32, 16)
    scb, ixb, ovb = (sc0, sc1), (ix0, ix1), (ov0, ov1)
    sib, sob = (si0, si1), (so0, so1)

    def start_in(g, b):
        p0 = p0w + g * _CH
        pltpu.make_async_copy(
            scores_hbm.at[pl.ds(p0, _CH)], scb[b], sib[b]).start()
        pltpu.make_async_copy(
            idx_hbm.at[pl.ds(2 * p0, 2 * _CH)], ixb[b], sib[b]).start()

    def wait_in(b):
        pltpu.make_async_copy(
            scores_hbm.at[pl.ds(0, _CH)], scb[b], sib[b]).wait()
        pltpu.make_async_copy(
            idx_hbm.at[pl.ds(0, 2 * _CH)], ixb[b], sib[b]).wait()

    def start_out(g, b):
        p0 = p0w + g * _CH
        pltpu.make_async_copy(
            ovb[b], out_hbm.at[pl.ds(2 * p0, 2 * _CH)], sob[b]).start()

    def wait_out(b):
        pltpu.make_async_copy(
            ovb[b], out_hbm.at[pl.ds(0, 2 * _CH)], sob[b]).wait()

    for b in range(2):
        start_in(b, b)

    def outer(g2, carry):
        for b in range(2):
            g = 2 * g2 + b

            @pl.when(g2 >= 1)
            def _():
                wait_out(b)

            wait_in(b)
            sc_v, idx_v, out_v = scb[b], ixb[b], ovb[b]
            for r in range(_CH):
                rr = jnp.full((16,), r, jnp.int32)
                for par in range(2):
                    qrow = jnp.full((16,), 2 * r + par, jnp.int32)
                    for half in range(2):
                        ce = jnp.full((16,), 32 * half, jnp.int32) + 2 * iota
                        ie = plsc.load_gather(idx_v, [qrow, ce])
                        io = plsc.load_gather(idx_v, [qrow, ce + 1])
                        ge = plsc.load_gather(sc_v, [rr, ie])
                        go = plsc.load_gather(sc_v, [rr, io])
                        if par == 0:
                            w = (ge & 0xFFFF) | lax.shift_left(go, 16)
                        else:
                            w = lax.shift_right_logical(ge, 16) | (
                                go & jnp.int32(-65536))
                        out_v[2 * r + par, pl.ds(16 * half, 16)] = w

            start_out(g, b)

            @pl.when(g2 < _NCHUNK // 2 - 1)
            def _():
                start_in(g + 2, b)

        return carry

    lax.fori_loop(0, _NCHUNK // 2, outer, 0)
    for b in range(2):
        wait_out(b)


def _sc_gather(scores, idx2):
    mesh = plsc.VectorSubcoreMesh(core_axis_name="c", subcore_axis_name="s")
    fn = functools.partial(
        pl.kernel,
        mesh=mesh,
        compiler_params=pltpu.CompilerParams(needs_layout_passes=False),
        out_type=jax.ShapeDtypeStruct((_RG, _K // 2), jnp.int32),
        scratch_types=[
            pltpu.VMEM((_CH, _S), jnp.int32),
            pltpu.VMEM((_CH, _S), jnp.int32),
            pltpu.VMEM((2 * _CH, _K), jnp.int32),
            pltpu.VMEM((2 * _CH, _K), jnp.int32),
            pltpu.VMEM((2 * _CH, _K // 2), jnp.int32),
            pltpu.VMEM((2 * _CH, _K // 2), jnp.int32),
            pltpu.SemaphoreType.DMA,
            pltpu.SemaphoreType.DMA,
            pltpu.SemaphoreType.DMA,
            pltpu.SemaphoreType.DMA,
        ],
    )(_sc_gather_body)
    return fn(scores, idx2)


@jax.jit
def kernel(q, k, idx):
    q3 = q.reshape(_H, _S, _D)
    k3 = k.reshape(_H, _S, _D)
    idx2 = idx.reshape(_R, _K)
    outs = []
    for g in range(_G):
        hsl = slice(g * _HG, (g + 1) * _HG)
        scores_g = _tc_scores(q3[hsl], k3[hsl])
        outs.append(_sc_gather(scores_g, idx2[g * _RG:(g + 1) * _RG]))
    out32 = jnp.concatenate(outs, axis=0)
    out = lax.bitcast_convert_type(out32, jnp.bfloat16)
    return out.reshape(_B, _H, _S, _K)
